# Initial kernel scaffold; baseline (speedup 1.0000x reference)
import functools

import jax
import jax.numpy as jnp
from jax import lax
from jax.experimental import pallas as pl
from jax.experimental.pallas import tpu as pltpu
from jax.experimental.pallas import tpu_sc as plsc

N = 10000
E = 320000
D_IN = 128
D_OUT = 9
L = 16

NC = 2
NS = 16
NW = NC * NS
CH = 128
NCHUNK = 80
E_PAD = NW * NCHUNK * CH
N_PAD = 10016
RPT = N_PAD // NS


def _proj_body(x_ref, wl_ref, wr_ref, bl_ref, y_ref, z_ref):
    x = x_ref[...]
    y = jnp.dot(x, wl_ref[...], preferred_element_type=jnp.float32)
    row = lax.broadcasted_iota(jnp.int32, (N_PAD, L), 0)
    col = lax.broadcasted_iota(jnp.int32, (N_PAD, L), 1)
    y_ref[...] = jnp.where((col == L - 1) & (row < N), 1.0, y)
    z_ref[...] = jnp.dot(x, wr_ref[...], preferred_element_type=jnp.float32) + bl_ref[...]


def _sc_body(y_hbm, src_hbm, dst_hbm, zero_hbm, out_hbm,
             src_v, dst_v, rows_v, acc_s, sem):
    cid = lax.axis_index("c")
    sid = lax.axis_index("s")
    tid = cid * NS + sid

    pltpu.sync_copy(zero_hbm.at[pl.ds(sid * RPT, RPT)],
                    acc_s.at[pl.ds(sid * RPT, RPT)])
    pltpu.sync_copy(src_hbm.at[tid], src_v)
    pltpu.sync_copy(dst_hbm.at[tid], dst_v)
    plsc.subcore_barrier()

    def body(j, carry):
        pltpu.async_copy(y_hbm.at[src_v.at[j]], rows_v, sem).wait()
        pltpu.sync_copy(rows_v, acc_s.at[dst_v.at[j]], add=True)
        return carry

    lax.fori_loop(0, NCHUNK, body, 0)
    plsc.subcore_barrier()
    pltpu.sync_copy(acc_s.at[pl.ds(sid * RPT, RPT)],
                    out_hbm.at[cid, pl.ds(sid * RPT, RPT)])


def _fin_body(p_ref, z_ref, o_ref):
    p = p_ref[0] + p_ref[1]
    cnt = p[:, L - 1:L]
    u = p / jnp.maximum(cnt, 1.0) + z_ref[...]
    col = lax.broadcasted_iota(jnp.int32, (N_PAD, L), 1)
    u = jnp.where(col < D_OUT, u, -jnp.inf)
    m = jnp.max(u, axis=1, keepdims=True)
    lse = jnp.log(jnp.sum(jnp.exp(u - m), axis=1, keepdims=True)) + m
    o_ref[...] = u - lse


def kernel(x, edge_index, W_l, b_l, W_r):
    f32 = jnp.float32
    src = edge_index[0].astype(jnp.int32)
    dst = edge_index[1].astype(jnp.int32)
    pad = jnp.full((E_PAD - E,), N, jnp.int32)
    src_g = jnp.concatenate([src, pad]).reshape(NW, NCHUNK, CH)
    dst_g = jnp.concatenate([dst, pad]).reshape(NW, NCHUNK, CH)

    x_pad = jnp.zeros((N_PAD, D_IN), f32).at[:N].set(x)
    wl_pad = jnp.zeros((D_IN, L), f32).at[:, :D_OUT].set(W_l)
    wr_pad = jnp.zeros((D_IN, L), f32).at[:, :D_OUT].set(W_r)
    bl_pad = jnp.zeros((1, L), f32).at[0, :D_OUT].set(b_l)

    y_pad, z_pad = pl.pallas_call(
        _proj_body,
        out_shape=[jax.ShapeDtypeStruct((N_PAD, L), f32),
                   jax.ShapeDtypeStruct((N_PAD, L), f32)],
    )(x_pad, wl_pad, wr_pad, bl_pad)

    sc_call = pl.kernel(
        _sc_body,
        out_type=jax.ShapeDtypeStruct((NC, N_PAD, L), f32),
        mesh=plsc.VectorSubcoreMesh(core_axis_name="c", subcore_axis_name="s"),
        scratch_types=[
            pltpu.VMEM((NCHUNK, CH), jnp.int32),
            pltpu.VMEM((NCHUNK, CH), jnp.int32),
            pltpu.VMEM((CH, L), f32),
            pltpu.VMEM_SHARED((N_PAD, L), f32),
            pltpu.SemaphoreType.DMA,
        ],
    )
    partials = sc_call(y_pad, src_g, dst_g, jnp.zeros((N_PAD, L), f32))

    out_pad = pl.pallas_call(
        _fin_body,
        out_shape=jax.ShapeDtypeStruct((N_PAD, L), f32),
    )(partials, z_pad)
    return out_pad[:N, :D_OUT]


# speedup vs baseline: 12.3846x; 12.3846x over previous
import functools

import jax
import jax.numpy as jnp
from jax import lax
from jax.experimental import pallas as pl
from jax.experimental.pallas import tpu as pltpu
from jax.experimental.pallas import tpu_sc as plsc

N = 10000
E = 320000
D_IN = 128
D_OUT = 9
L = 16

NC = 2
NS = 16
NW = NC * NS
CH = 128
NCHUNK = 80
E_PAD = NW * NCHUNK * CH
N_PAD = 10112
RPT = N_PAD // NS


def _proj_body(x_ref, wl_ref, wr_ref, bl_ref, y_ref, z_ref):
    x = x_ref[...]
    y = jnp.dot(x, wl_ref[...], preferred_element_type=jnp.float32)
    row = lax.broadcasted_iota(jnp.int32, (N_PAD, L), 0)
    col = lax.broadcasted_iota(jnp.int32, (N_PAD, L), 1)
    y_ref[...] = jnp.where((col == L - 1) & (row < N), 1.0, y)
    z_ref[...] = jnp.dot(x, wr_ref[...], preferred_element_type=jnp.float32) + bl_ref[...]


def _sc_body(y_hbm, src_hbm, dst_hbm, zero_hbm, out_hbm,
             src_v, dst_v, rows_v, acc_s, sem):
    cid = lax.axis_index("c")
    sid = lax.axis_index("s")
    tid = cid * NS + sid

    pltpu.sync_copy(zero_hbm.at[pl.ds(sid * RPT, RPT)],
                    acc_s.at[pl.ds(sid * RPT, RPT)])
    pltpu.sync_copy(src_hbm.at[tid], src_v)
    pltpu.sync_copy(dst_hbm.at[tid], dst_v)
    plsc.subcore_barrier()

    def body(j, carry):
        pltpu.async_copy(y_hbm.at[src_v.at[j]], rows_v, sem).wait()
        pltpu.sync_copy(rows_v, acc_s.at[dst_v.at[j]], add=True)
        return carry

    lax.fori_loop(0, NCHUNK, body, 0)
    plsc.subcore_barrier()
    pltpu.sync_copy(acc_s.at[pl.ds(sid * RPT, RPT)],
                    out_hbm.at[cid, pl.ds(sid * RPT, RPT)])


def _fin_body(p_ref, z_ref, o_ref):
    p = p_ref[0] + p_ref[1]
    cnt = p[:, L - 1:L]
    u = p / jnp.maximum(cnt, 1.0) + z_ref[...]
    col = lax.broadcasted_iota(jnp.int32, (N_PAD, L), 1)
    u = jnp.where(col < D_OUT, u, -jnp.inf)
    m = jnp.max(u, axis=1, keepdims=True)
    lse = jnp.log(jnp.sum(jnp.exp(u - m), axis=1, keepdims=True)) + m
    o_ref[...] = u - lse


def kernel(x, edge_index, W_l, b_l, W_r):
    f32 = jnp.float32
    src = edge_index[0].astype(jnp.int32)
    dst = edge_index[1].astype(jnp.int32)
    pad = jnp.full((E_PAD - E,), N, jnp.int32)
    src_g = jnp.concatenate([src, pad]).reshape(NW, NCHUNK, CH)
    dst_g = jnp.concatenate([dst, pad]).reshape(NW, NCHUNK, CH)

    x_pad = jnp.zeros((N_PAD, D_IN), f32).at[:N].set(x)
    wl_pad = jnp.zeros((D_IN, L), f32).at[:, :D_OUT].set(W_l)
    wr_pad = jnp.zeros((D_IN, L), f32).at[:, :D_OUT].set(W_r)
    bl_pad = jnp.zeros((1, L), f32).at[0, :D_OUT].set(b_l)

    y_pad, z_pad = pl.pallas_call(
        _proj_body,
        out_shape=[jax.ShapeDtypeStruct((N_PAD, L), f32),
                   jax.ShapeDtypeStruct((N_PAD, L), f32)],
    )(x_pad, wl_pad, wr_pad, bl_pad)

    sc_call = pl.kernel(
        _sc_body,
        out_type=jax.ShapeDtypeStruct((NC, N_PAD, L), f32),
        mesh=plsc.VectorSubcoreMesh(core_axis_name="c", subcore_axis_name="s"),
        compiler_params=pltpu.CompilerParams(use_tc_tiling_on_sc=False),
        scratch_types=[
            pltpu.VMEM((NCHUNK, CH), jnp.int32),
            pltpu.VMEM((NCHUNK, CH), jnp.int32),
            pltpu.VMEM((CH, L), f32),
            pltpu.VMEM_SHARED((N_PAD, L), f32),
            pltpu.SemaphoreType.DMA,
        ],
    )
    partials = sc_call(y_pad, src_g, dst_g, jnp.zeros((N_PAD, L), f32))

    out_pad = pl.pallas_call(
        _fin_body,
        out_shape=jax.ShapeDtypeStruct((N_PAD, L), f32),
    )(partials, z_pad)
    return out_pad[:N, :D_OUT]


# speedup vs baseline: 13.1679x; 1.0632x over previous
import functools

import jax
import jax.numpy as jnp
from jax import lax
from jax.experimental import pallas as pl
from jax.experimental.pallas import tpu as pltpu
from jax.experimental.pallas import tpu_sc as plsc

N = 10000
E = 320000
D_IN = 128
D_OUT = 9
L = 16

NC = 2
NS = 16
NW = NC * NS
CH = 128
NCHUNK = 80
E_PAD = NW * NCHUNK * CH
N_PAD = 10112
RPT = N_PAD // NS


def _proj_body(x_ref, wl_ref, wr_ref, bl_ref, y_ref, z_ref):
    x = x_ref[...]
    y = jnp.dot(x, wl_ref[...], preferred_element_type=jnp.float32)
    row = lax.broadcasted_iota(jnp.int32, (N_PAD, L), 0)
    col = lax.broadcasted_iota(jnp.int32, (N_PAD, L), 1)
    y_ref[...] = jnp.where((col == L - 1) & (row < N), 1.0, y)
    z_ref[...] = jnp.dot(x, wr_ref[...], preferred_element_type=jnp.float32) + bl_ref[...]


def _sc_body(y_hbm, src_hbm, dst_hbm, zero_hbm, out_hbm,
             src_v, dst_v, rows_a, rows_b, acc_s, sem_a, sem_b):
    cid = lax.axis_index("c")
    sid = lax.axis_index("s")
    tid = cid * NS + sid

    pltpu.sync_copy(zero_hbm.at[pl.ds(sid * RPT, RPT)],
                    acc_s.at[pl.ds(sid * RPT, RPT)])
    pltpu.sync_copy(src_hbm.at[tid], src_v)
    pltpu.sync_copy(dst_hbm.at[tid], dst_v)
    plsc.subcore_barrier()

    def gather(j, buf, sem):
        return pltpu.async_copy(y_hbm.at[src_v.at[j]], buf, sem)

    def scatter(j, buf):
        pltpu.sync_copy(buf, acc_s.at[dst_v.at[j]], add=True)

    gather(0, rows_a, sem_a)

    def body(i, carry):
        j = 2 * i
        pltpu.make_async_copy(y_hbm.at[src_v.at[j]], rows_a, sem_a).wait()
        gather(j + 1, rows_b, sem_b)
        scatter(j, rows_a)
        pltpu.make_async_copy(y_hbm.at[src_v.at[j]], rows_b, sem_b).wait()
        gather(j + 2, rows_a, sem_a)
        scatter(j + 1, rows_b)
        return carry

    lax.fori_loop(0, NCHUNK // 2 - 1, body, 0)
    j = NCHUNK - 2
    pltpu.make_async_copy(y_hbm.at[src_v.at[j]], rows_a, sem_a).wait()
    gather(j + 1, rows_b, sem_b)
    scatter(j, rows_a)
    pltpu.make_async_copy(y_hbm.at[src_v.at[j]], rows_b, sem_b).wait()
    scatter(j + 1, rows_b)
    plsc.subcore_barrier()
    pltpu.sync_copy(acc_s.at[pl.ds(sid * RPT, RPT)],
                    out_hbm.at[cid, pl.ds(sid * RPT, RPT)])


def _fin_body(p_ref, z_ref, o_ref):
    p = p_ref[0] + p_ref[1]
    cnt = p[:, L - 1:L]
    u = p / jnp.maximum(cnt, 1.0) + z_ref[...]
    col = lax.broadcasted_iota(jnp.int32, (N_PAD, L), 1)
    u = jnp.where(col < D_OUT, u, -jnp.inf)
    m = jnp.max(u, axis=1, keepdims=True)
    lse = jnp.log(jnp.sum(jnp.exp(u - m), axis=1, keepdims=True)) + m
    o_ref[...] = u - lse


def kernel(x, edge_index, W_l, b_l, W_r):
    f32 = jnp.float32
    src = edge_index[0].astype(jnp.int32)
    dst = edge_index[1].astype(jnp.int32)
    pad = jnp.full((E_PAD - E,), N, jnp.int32)
    src_g = jnp.concatenate([src, pad]).reshape(NW, NCHUNK, CH)
    dst_g = jnp.concatenate([dst, pad]).reshape(NW, NCHUNK, CH)

    x_pad = jnp.zeros((N_PAD, D_IN), f32).at[:N].set(x)
    wl_pad = jnp.zeros((D_IN, L), f32).at[:, :D_OUT].set(W_l)
    wr_pad = jnp.zeros((D_IN, L), f32).at[:, :D_OUT].set(W_r)
    bl_pad = jnp.zeros((1, L), f32).at[0, :D_OUT].set(b_l)

    y_pad, z_pad = pl.pallas_call(
        _proj_body,
        out_shape=[jax.ShapeDtypeStruct((N_PAD, L), f32),
                   jax.ShapeDtypeStruct((N_PAD, L), f32)],
    )(x_pad, wl_pad, wr_pad, bl_pad)

    sc_call = pl.kernel(
        _sc_body,
        out_type=jax.ShapeDtypeStruct((NC, N_PAD, L), f32),
        mesh=plsc.VectorSubcoreMesh(core_axis_name="c", subcore_axis_name="s"),
        compiler_params=pltpu.CompilerParams(use_tc_tiling_on_sc=False),
        scratch_types=[
            pltpu.VMEM((NCHUNK, CH), jnp.int32),
            pltpu.VMEM((NCHUNK, CH), jnp.int32),
            pltpu.VMEM((CH, L), f32),
            pltpu.VMEM((CH, L), f32),
            pltpu.VMEM_SHARED((N_PAD, L), f32),
            pltpu.SemaphoreType.DMA,
            pltpu.SemaphoreType.DMA,
        ],
    )
    partials = sc_call(y_pad, src_g, dst_g, jnp.zeros((N_PAD, L), f32))

    out_pad = pl.pallas_call(
        _fin_body,
        out_shape=jax.ShapeDtypeStruct((N_PAD, L), f32),
    )(partials, z_pad)
    return out_pad[:N, :D_OUT]


# speedup vs baseline: 15.4299x; 1.1718x over previous
import functools

import jax
import jax.numpy as jnp
from jax import lax
from jax.experimental import pallas as pl
from jax.experimental.pallas import tpu as pltpu
from jax.experimental.pallas import tpu_sc as plsc

N = 10000
E = 320000
D_IN = 128
D_OUT = 9
L = 16

NC = 2
NS = 16
NW = NC * NS
CH = 128
NCHUNK = 80
E_PAD = NW * NCHUNK * CH
N_PAD = 10112
RPT = N_PAD // NS


def _proj_body(x_ref, wl_ref, wr_ref, bl_ref, y_ref, z_ref):
    x = x_ref[...]
    y = jnp.dot(x, wl_ref[...], preferred_element_type=jnp.float32)
    row = lax.broadcasted_iota(jnp.int32, (N_PAD, L), 0)
    col = lax.broadcasted_iota(jnp.int32, (N_PAD, L), 1)
    y_ref[...] = jnp.where((col == L - 1) & (row < N), 1.0, y)
    z_ref[...] = jnp.dot(x, wr_ref[...], preferred_element_type=jnp.float32) + bl_ref[...]


NBUF = 4


def _sc_body(y_hbm, src_hbm, dst_hbm, zero_hbm, out_hbm,
             src_v, dst_v, bufs, acc_s, gsems, ssems):
    cid = lax.axis_index("c")
    sid = lax.axis_index("s")
    tid = cid * NS + sid

    pltpu.sync_copy(zero_hbm.at[pl.ds(sid * RPT, RPT)],
                    acc_s.at[pl.ds(sid * RPT, RPT)])
    pltpu.sync_copy(src_hbm.at[tid], src_v)
    pltpu.sync_copy(dst_hbm.at[tid], dst_v)
    plsc.subcore_barrier()

    def gather(j, b):
        pltpu.async_copy(y_hbm.at[src_v.at[j]], bufs[b], gsems[b])

    def wait_gather(j, b):
        pltpu.make_async_copy(y_hbm.at[src_v.at[j]], bufs[b], gsems[b]).wait()

    def scatter(j, b):
        return pltpu.async_copy(bufs[b], acc_s.at[dst_v.at[j]], ssems[b],
                                add=True)

    for b in range(NBUF):
        gather(b, b)

    def body(i, carry):
        descs = []
        for b in range(NBUF):
            j = NBUF * i + b
            wait_gather(j, b)
            descs.append(scatter(j, b))
        for b in range(NBUF):
            descs[b].wait()
            gather(NBUF * i + NBUF + b, b)
        return carry

    lax.fori_loop(0, NCHUNK // NBUF - 1, body, 0)
    tail = []
    for b in range(NBUF):
        j = NCHUNK - NBUF + b
        wait_gather(j, b)
        tail.append(scatter(j, b))
    for d in tail:
        d.wait()
    plsc.subcore_barrier()
    pltpu.sync_copy(acc_s.at[pl.ds(sid * RPT, RPT)],
                    out_hbm.at[cid, pl.ds(sid * RPT, RPT)])


def _fin_body(p_ref, z_ref, o_ref):
    p = p_ref[0] + p_ref[1]
    cnt = p[:, L - 1:L]
    u = p / jnp.maximum(cnt, 1.0) + z_ref[...]
    col = lax.broadcasted_iota(jnp.int32, (N_PAD, L), 1)
    u = jnp.where(col < D_OUT, u, -jnp.inf)
    m = jnp.max(u, axis=1, keepdims=True)
    lse = jnp.log(jnp.sum(jnp.exp(u - m), axis=1, keepdims=True)) + m
    o_ref[...] = u - lse


def kernel(x, edge_index, W_l, b_l, W_r):
    f32 = jnp.float32
    src = edge_index[0].astype(jnp.int32)
    dst = edge_index[1].astype(jnp.int32)
    pad = jnp.full((E_PAD - E,), N, jnp.int32)
    src_g = jnp.concatenate([src, pad]).reshape(NW, NCHUNK, CH)
    dst_g = jnp.concatenate([dst, pad]).reshape(NW, NCHUNK, CH)

    x_pad = jnp.zeros((N_PAD, D_IN), f32).at[:N].set(x)
    wl_pad = jnp.zeros((D_IN, L), f32).at[:, :D_OUT].set(W_l)
    wr_pad = jnp.zeros((D_IN, L), f32).at[:, :D_OUT].set(W_r)
    bl_pad = jnp.zeros((1, L), f32).at[0, :D_OUT].set(b_l)

    y_pad, z_pad = pl.pallas_call(
        _proj_body,
        out_shape=[jax.ShapeDtypeStruct((N_PAD, L), f32),
                   jax.ShapeDtypeStruct((N_PAD, L), f32)],
    )(x_pad, wl_pad, wr_pad, bl_pad)

    sc_call = pl.kernel(
        _sc_body,
        out_type=jax.ShapeDtypeStruct((NC, N_PAD, L), f32),
        mesh=plsc.VectorSubcoreMesh(core_axis_name="c", subcore_axis_name="s"),
        compiler_params=pltpu.CompilerParams(use_tc_tiling_on_sc=False),
        scratch_types=[
            pltpu.VMEM((NCHUNK, CH), jnp.int32),
            pltpu.VMEM((NCHUNK, CH), jnp.int32),
            [pltpu.VMEM((CH, L), f32)] * NBUF,
            pltpu.VMEM_SHARED((N_PAD, L), f32),
            [pltpu.SemaphoreType.DMA] * NBUF,
            [pltpu.SemaphoreType.DMA] * NBUF,
        ],
    )
    partials = sc_call(y_pad, src_g, dst_g, jnp.zeros((N_PAD, L), f32))

    out_pad = pl.pallas_call(
        _fin_body,
        out_shape=jax.ShapeDtypeStruct((N_PAD, L), f32),
    )(partials, z_pad)
    return out_pad[:N, :D_OUT]


# speedup vs baseline: 21.2704x; 1.3785x over previous
import functools

import jax
import jax.numpy as jnp
from jax import lax
from jax.experimental import pallas as pl
from jax.experimental.pallas import tpu as pltpu
from jax.experimental.pallas import tpu_sc as plsc

N = 10000
E = 320000
D_IN = 128
D_OUT = 9
L = 16

NC = 2
NS = 16
NW = NC * NS
CH = 128
NCHUNK = 80
E_PAD = NW * NCHUNK * CH
N_PAD = 10112
RPT = N_PAD // NS


def _proj_body(x_ref, wl_ref, wr_ref, bl_ref, y_ref, z_ref):
    x = x_ref[...]
    y = jnp.dot(x, wl_ref[...], preferred_element_type=jnp.float32)
    row = lax.broadcasted_iota(jnp.int32, (N_PAD, L), 0)
    col = lax.broadcasted_iota(jnp.int32, (N_PAD, L), 1)
    y_ref[...] = jnp.where((col == L - 1) & (row < N), 1.0, y)
    z_ref[...] = jnp.dot(x, wr_ref[...], preferred_element_type=jnp.float32) + bl_ref[...]


NBUF = 4


def _sc_body(y_hbm, src_hbm, dst_hbm, zero_hbm, out_hbm,
             src_v, dst_v, bufs, y_s, acc_s, gsems, ssems):
    cid = lax.axis_index("c")
    sid = lax.axis_index("s")
    tid = cid * NS + sid

    pltpu.sync_copy(zero_hbm.at[pl.ds(sid * RPT, RPT)],
                    acc_s.at[pl.ds(sid * RPT, RPT)])
    pltpu.sync_copy(y_hbm.at[pl.ds(sid * RPT, RPT)],
                    y_s.at[pl.ds(sid * RPT, RPT)])
    pltpu.sync_copy(src_hbm.at[tid], src_v)
    pltpu.sync_copy(dst_hbm.at[tid], dst_v)
    plsc.subcore_barrier()

    def gather(j, b):
        pltpu.async_copy(y_s.at[src_v.at[j]], bufs[b], gsems[b])

    def wait_gather(j, b):
        pltpu.make_async_copy(y_s.at[src_v.at[j]], bufs[b], gsems[b]).wait()

    def scatter(j, b):
        return pltpu.async_copy(bufs[b], acc_s.at[dst_v.at[j]], ssems[b],
                                add=True)

    for b in range(NBUF):
        gather(b, b)

    def body(i, carry):
        descs = []
        for b in range(NBUF):
            j = NBUF * i + b
            wait_gather(j, b)
            descs.append(scatter(j, b))
        for b in range(NBUF):
            descs[b].wait()
            gather(NBUF * i + NBUF + b, b)
        return carry

    lax.fori_loop(0, NCHUNK // NBUF - 1, body, 0)
    tail = []
    for b in range(NBUF):
        j = NCHUNK - NBUF + b
        wait_gather(j, b)
        tail.append(scatter(j, b))
    for d in tail:
        d.wait()
    plsc.subcore_barrier()
    pltpu.sync_copy(acc_s.at[pl.ds(sid * RPT, RPT)],
                    out_hbm.at[cid, pl.ds(sid * RPT, RPT)])


def _fin_body(p_ref, z_ref, o_ref):
    p = p_ref[0] + p_ref[1]
    cnt = p[:, L - 1:L]
    u = p / jnp.maximum(cnt, 1.0) + z_ref[...]
    col = lax.broadcasted_iota(jnp.int32, (N_PAD, L), 1)
    u = jnp.where(col < D_OUT, u, -jnp.inf)
    m = jnp.max(u, axis=1, keepdims=True)
    lse = jnp.log(jnp.sum(jnp.exp(u - m), axis=1, keepdims=True)) + m
    o_ref[...] = u - lse


def kernel(x, edge_index, W_l, b_l, W_r):
    f32 = jnp.float32
    src = edge_index[0].astype(jnp.int32)
    dst = edge_index[1].astype(jnp.int32)
    pad = jnp.full((E_PAD - E,), N, jnp.int32)
    src_g = jnp.concatenate([src, pad]).reshape(NW, NCHUNK, CH)
    dst_g = jnp.concatenate([dst, pad]).reshape(NW, NCHUNK, CH)

    x_pad = jnp.zeros((N_PAD, D_IN), f32).at[:N].set(x)
    wl_pad = jnp.zeros((D_IN, L), f32).at[:, :D_OUT].set(W_l)
    wr_pad = jnp.zeros((D_IN, L), f32).at[:, :D_OUT].set(W_r)
    bl_pad = jnp.zeros((1, L), f32).at[0, :D_OUT].set(b_l)

    y_pad, z_pad = pl.pallas_call(
        _proj_body,
        out_shape=[jax.ShapeDtypeStruct((N_PAD, L), f32),
                   jax.ShapeDtypeStruct((N_PAD, L), f32)],
    )(x_pad, wl_pad, wr_pad, bl_pad)

    sc_call = pl.kernel(
        _sc_body,
        out_type=jax.ShapeDtypeStruct((NC, N_PAD, L), f32),
        mesh=plsc.VectorSubcoreMesh(core_axis_name="c", subcore_axis_name="s"),
        compiler_params=pltpu.CompilerParams(use_tc_tiling_on_sc=False),
        scratch_types=[
            pltpu.VMEM((NCHUNK, CH), jnp.int32),
            pltpu.VMEM((NCHUNK, CH), jnp.int32),
            [pltpu.VMEM((CH, L), f32)] * NBUF,
            pltpu.VMEM_SHARED((N_PAD, L), f32),
            pltpu.VMEM_SHARED((N_PAD, L), f32),
            [pltpu.SemaphoreType.DMA] * NBUF,
            [pltpu.SemaphoreType.DMA] * NBUF,
        ],
    )
    partials = sc_call(y_pad, src_g, dst_g, jnp.zeros((N_PAD, L), f32))

    out_pad = pl.pallas_call(
        _fin_body,
        out_shape=jax.ShapeDtypeStruct((N_PAD, L), f32),
    )(partials, z_pad)
    return out_pad[:N, :D_OUT]
